# Initial kernel scaffold; baseline (speedup 1.0000x reference)
import jax
import jax.numpy as jnp
from jax.experimental import pallas as pl


def kernel(x, coordinates, W1, b1, g1, be1, W2, b2, g2, be2, W3, b3, g3, be3):
    raise NotImplementedError("write your pallas kernel here")


# speedup vs baseline: 10.9642x; 10.9642x over previous
import functools
import math

import jax
import jax.numpy as jnp
from jax import lax
from jax.experimental import pallas as pl
from jax.experimental.pallas import tpu as pltpu
from jax.experimental.pallas import tpu_sc as plsc

K = 16
M1 = 256
M3 = 128


def _gelu(v):
    return v * 0.5 * (1.0 + lax.erf(v * (1.0 / math.sqrt(2.0))))


def _ln(h, g, b, eps=1e-5):
    mu = jnp.mean(h, axis=-1, keepdims=True)
    var = jnp.mean((h - mu) ** 2, axis=-1, keepdims=True)
    return (h - mu) / jnp.sqrt(var + eps) * g + b


def _bf16_round(v):
    b = lax.bitcast_convert_type(v, jnp.uint32)
    r = (b + jnp.uint32(0x7FFF) + ((b >> 16) & jnp.uint32(1))) \
        & jnp.uint32(0xFFFF0000)
    return lax.bitcast_convert_type(r, jnp.float32)


def _topk_q_body(coord_ref, coordt_ref, x_ref, w1b_ref, gidx_ref, q_ref):
    b = pl.program_id(0)
    n = coordt_ref.shape[2]
    c = coord_ref[0]
    ct = coordt_ref[0]
    cb = _bf16_round(c)
    ctb = _bf16_round(ct)
    cxi, cyi, czi = c[:, 0:1], c[:, 1:2], c[:, 2:3]
    cxj, cyj, czj = ct[0:1, :], ct[1:2, :], ct[2:3, :]
    bxi, byi, bzi = cb[:, 0:1], cb[:, 1:2], cb[:, 2:3]
    bxj, byj, bzj = ctb[0:1, :], ctb[1:2, :], ctb[2:3, :]
    sqi = cxi * cxi + cyi * cyi + czi * czi
    sqj = cxj * cxj + cyj * cyj + czj * czj
    dot = bxi * bxj + byi * byj + bzi * bzj
    d2 = sqi + sqj - 2.0 * dot
    iota = lax.broadcasted_iota(jnp.int32, d2.shape, 1)
    cols = []
    for _ in range(K):
        m = jnp.min(d2, axis=1, keepdims=True)
        j = jnp.min(jnp.where(d2 <= m, iota, n), axis=1, keepdims=True)
        cols.append(j)
        d2 = jnp.where(iota == j, jnp.inf, d2)
    idx = jnp.concatenate(cols, axis=1)
    gidx_ref[0] = idx + b * n
    q_ref[0] = jnp.dot(x_ref[0], w1b_ref[...],
                       preferred_element_type=jnp.float32)


def _mlp_body(qg_ref, x_ref, w1d_ref, b1_ref, g1_ref, be1_ref,
              w2_ref, b2_ref, g2_ref, be2_ref,
              w3_ref, b3_ref, g3_ref, be3_ref, out_ref):
    xb = x_ref[0]
    e = w1d_ref.shape[1]
    p = jnp.dot(xb, w1d_ref[...], preferred_element_type=jnp.float32)
    p = p + b1_ref[...]
    qg = qg_ref[0]
    t = qg.reshape(M3, K, e) + p[:, None, :]
    h = _gelu(_ln(t.reshape(M3 * K, e), g1_ref[...], be1_ref[...]))
    h = jnp.dot(h, w2_ref[...], preferred_element_type=jnp.float32)
    h = _gelu(_ln(h + b2_ref[...], g2_ref[...], be2_ref[...]))
    h = jnp.dot(h, w3_ref[...], preferred_element_type=jnp.float32)
    h = _gelu(_ln(h + b3_ref[...], g3_ref[...], be3_ref[...]))
    agg = jnp.max(h.reshape(M3, K, e), axis=1)
    out_ref[0] = _gelu(agg + xb)


def _sc_gather(table, gidx, n_rows, d, chunk=128):
    total = gidx.shape[0]
    info = plsc.get_sparse_core_info()
    nc, ns = info.num_cores, info.num_subcores
    nw = nc * ns
    per_w = total // nw
    n_chunks = per_w // chunk
    mesh = plsc.VectorSubcoreMesh(core_axis_name="c", subcore_axis_name="s")

    @functools.partial(
        pl.kernel, mesh=mesh,
        compiler_params=pltpu.CompilerParams(use_tc_tiling_on_sc=False),
        out_type=jax.ShapeDtypeStruct((total, d), jnp.float32),
        scratch_types=[
            pltpu.VMEM((chunk,), jnp.int32),
            pltpu.VMEM((chunk, d), jnp.float32),
            pltpu.SemaphoreType.DMA,
        ],
    )
    def gather_k(table_hbm, idx_hbm, out_hbm, idx_v, rows_v, sem):
        wid = lax.axis_index("s") * nc + lax.axis_index("c")
        w_base = wid * per_w

        def body(ci, _):
            base = w_base + ci * chunk
            pltpu.sync_copy(idx_hbm.at[pl.ds(base, chunk)], idx_v)
            pltpu.async_copy(table_hbm.at[idx_v], rows_v, sem).wait()
            pltpu.sync_copy(rows_v, out_hbm.at[pl.ds(base, chunk)])
            return 0

        lax.fori_loop(0, n_chunks, body, 0)

    return gather_k(table, gidx)


def kernel(x, coordinates, W1, b1, g1, be1, W2, b2, g2, be2, W3, b3, g3, be3):
    B, N, D = x.shape
    E = W1.shape[1]
    coordsT = jnp.transpose(coordinates, (0, 2, 1))
    W1a, W1b = W1[:D], W1[D:]
    W1d = W1a - W1b

    gidx, Q = pl.pallas_call(
        _topk_q_body,
        grid=(B, N // M1),
        in_specs=[
            pl.BlockSpec((1, M1, 3), lambda b, i: (b, i, 0)),
            pl.BlockSpec((1, 3, N), lambda b, i: (b, 0, 0)),
            pl.BlockSpec((1, M1, D), lambda b, i: (b, i, 0)),
            pl.BlockSpec((D, E), lambda b, i: (0, 0)),
        ],
        out_specs=[
            pl.BlockSpec((1, M1, K), lambda b, i: (b, i, 0)),
            pl.BlockSpec((1, M1, E), lambda b, i: (b, i, 0)),
        ],
        out_shape=[
            jax.ShapeDtypeStruct((B, N, K), jnp.int32),
            jax.ShapeDtypeStruct((B, N, E), jnp.float32),
        ],
    )(coordinates, coordsT, x, W1b)

    qg = _sc_gather(Q.reshape(B * N, E), gidx.reshape(B * N * K), B * N, E)
    qg = qg.reshape(B, N * K, E)

    vec = lambda v: v.reshape(1, E)
    out = pl.pallas_call(
        _mlp_body,
        grid=(B, N // M3),
        in_specs=[
            pl.BlockSpec((1, M3 * K, E), lambda b, i: (b, i, 0)),
            pl.BlockSpec((1, M3, D), lambda b, i: (b, i, 0)),
            pl.BlockSpec((D, E), lambda b, i: (0, 0)),
        ] + [pl.BlockSpec((1, E), lambda b, i: (0, 0))] * 3 + [
            pl.BlockSpec((E, E), lambda b, i: (0, 0)),
        ] + [pl.BlockSpec((1, E), lambda b, i: (0, 0))] * 3 + [
            pl.BlockSpec((E, E), lambda b, i: (0, 0)),
        ] + [pl.BlockSpec((1, E), lambda b, i: (0, 0))] * 3,
        out_specs=pl.BlockSpec((1, M3, D), lambda b, i: (b, i, 0)),
        out_shape=jax.ShapeDtypeStruct((B, N, D), jnp.float32),
    )(qg, x, W1d, vec(b1), vec(g1), vec(be1),
      W2, vec(b2), vec(g2), vec(be2),
      W3, vec(b3), vec(g3), vec(be3))
    return out
